# baseline (speedup 1.0000x reference)
import jax
import jax.numpy as jnp
from jax import lax
from jax.experimental import pallas as pl
from jax.experimental.pallas import tpu as pltpu
from jax.experimental.pallas import tpu_sc as plsc

M, N, D = 8192, 4, 2048
NC, NS = 2, 16
NW = NC * NS
RW = M // NW
T = 8
NCHUNK = RW // T
VEC = 32


def _sc_body(res_hbm, wv_hbm, out_hbm,
             res_v0, res_v1, wv_v0, wv_v1, out_v0, out_v1,
             sem_in0, sem_in1, sem_w0, sem_w1, sem_out0, sem_out1):
    res_bufs = (res_v0, res_v1)
    wv_bufs = (wv_v0, wv_v1)
    out_bufs = (out_v0, out_v1)
    sems_in = (sem_in0, sem_in1)
    sems_w = (sem_w0, sem_w1)
    sems_out = (sem_out0, sem_out1)
    wid = lax.axis_index("s") * NC + lax.axis_index("c")
    row0 = wid * RW

    def load(k, buf):
        pltpu.async_copy(
            res_hbm.at[pl.ds(row0 + k * T, T)], res_bufs[buf], sems_in[buf])
        pltpu.async_copy(
            wv_hbm.at[pl.ds((row0 + k * T) * N, T * N)], wv_bufs[buf],
            sems_w[buf])

    def store(k, buf):
        pltpu.async_copy(
            out_bufs[buf], out_hbm.at[pl.ds(row0 + k * T, T)], sems_out[buf])

    def wait_load(buf):
        pltpu.make_async_copy(
            res_hbm.at[pl.ds(row0, T)], res_bufs[buf], sems_in[buf]).wait()
        pltpu.make_async_copy(
            wv_hbm.at[pl.ds(row0 * N, T * N)], wv_bufs[buf],
            sems_w[buf]).wait()

    def wait_store(buf):
        pltpu.make_async_copy(
            out_bufs[buf], out_hbm.at[pl.ds(row0, T)], sems_out[buf]).wait()

    def compute(k, buf):
        rv = res_bufs[buf]
        wv = wv_bufs[buf]
        ov = out_bufs[buf]

        @pl.loop(0, T)
        def _(t):
            w = [wv[t * N + n, :] for n in range(N)]
            for col in range(0, D, VEC):
                r = [rv[t, n, pl.ds(col, VEC)] for n in range(N)]
                acc = (r[0] * w[0] + r[1] * w[1]) + (r[2] * w[2] + r[3] * w[3])
                ov[t, pl.ds(col, VEC)] = acc

    load(0, 0)

    @pl.loop(0, NCHUNK, step=2)
    def _(k):
        for b in range(2):
            kk = k + b

            @pl.when(kk + 1 < NCHUNK)
            def _():
                load(kk + 1, 1 - b)

            wait_load(b)

            @pl.when(kk >= 2)
            def _():
                wait_store(b)

            compute(kk, b)
            store(kk, b)

    for b in range(2):
        wait_store(b)


def kernel(res, h_pre):
    wbc = jnp.broadcast_to(
        h_pre.astype(jnp.bfloat16)[:, :, None], (M, N, VEC)).reshape(M * N, VEC)

    mesh = plsc.VectorSubcoreMesh(core_axis_name="c", subcore_axis_name="s")
    f = pl.kernel(
        _sc_body,
        out_type=jax.ShapeDtypeStruct((M, D), jnp.bfloat16),
        mesh=mesh,
        compiler_params=pltpu.CompilerParams(
            needs_layout_passes=False, use_tc_tiling_on_sc=False),
        scratch_types=[
            pltpu.VMEM((T, N, D), jnp.bfloat16),
            pltpu.VMEM((T, N, D), jnp.bfloat16),
            pltpu.VMEM((T * N, VEC), jnp.bfloat16),
            pltpu.VMEM((T * N, VEC), jnp.bfloat16),
            pltpu.VMEM((T, D), jnp.bfloat16),
            pltpu.VMEM((T, D), jnp.bfloat16),
            pltpu.SemaphoreType.DMA,
            pltpu.SemaphoreType.DMA,
            pltpu.SemaphoreType.DMA,
            pltpu.SemaphoreType.DMA,
            pltpu.SemaphoreType.DMA,
            pltpu.SemaphoreType.DMA,
        ],
    )
    return f(res, wbc)


# speedup vs baseline: 1.2055x; 1.2055x over previous
import jax
import jax.numpy as jnp
from jax import lax
from jax.experimental import pallas as pl
from jax.experimental.pallas import tpu as pltpu
from jax.experimental.pallas import tpu_sc as plsc

M, N, D = 8192, 4, 2048
NC, NS = 2, 16
NW = NC * NS
RW = M // NW
T = 8
NCHUNK = RW // T
CT = D // 128
WPR = 2 * CT
OPR = CT * T // 2
FMT = plsc.PackFormat.INTERLEAVED


def _sc_body(res_hbm, wq_hbm, out_hbm, wq_v,
             res_v0, res_v1, out_v0, out_v1,
             sem_w, sem_in0, sem_in1, sem_out0, sem_out1):
    res_bufs = (res_v0, res_v1)
    out_bufs = (out_v0, out_v1)
    sems_in = (sem_in0, sem_in1)
    sems_out = (sem_out0, sem_out1)
    wid = lax.axis_index("s") * NC + lax.axis_index("c")
    row0 = wid * RW

    pltpu.async_copy(wq_hbm.at[pl.ds(row0 // 64, RW // 64)], wq_v, sem_w).wait()

    def load(k, buf):
        return pltpu.async_copy(
            res_hbm.at[pl.ds(row0 + k * T, T)],
            res_bufs[buf], sems_in[buf])

    def store(k, buf):
        return pltpu.async_copy(
            out_bufs[buf],
            out_hbm.at[pl.ds((row0 + k * T) * 8, OPR)], sems_out[buf])

    def wait_load(buf):
        pltpu.make_async_copy(
            res_hbm.at[pl.ds(row0, T)], res_bufs[buf],
            sems_in[buf]).wait()

    def wait_store(buf):
        pltpu.make_async_copy(
            out_bufs[buf], out_hbm.at[pl.ds(row0 * 8, OPR)],
            sems_out[buf]).wait()

    def compute(k, buf):
        rv = res_bufs[buf]
        ov = out_bufs[buf]
        wrow = (k * T) // 64
        wcol = 2 * ((k * T) % 64)
        wv16 = wq_v[wrow, pl.ds(wcol, 16)]
        wA = [plsc.bitcast(jnp.full((16,), 1, jnp.int32) * wv16[2 * t],
                           jnp.bfloat16) for t in range(T)]
        wB = [plsc.bitcast(jnp.full((16,), 1, jnp.int32) * wv16[2 * t + 1],
                           jnp.bfloat16) for t in range(T)]

        @pl.loop(0, CT)
        def _(c):
            for u in range(T // 2):
                for j in range(8):
                    o = [None, None]
                    for p in range(2):
                        t = 2 * u + p
                        a = plsc.bitcast(
                            rv[t, 0, c, pl.ds(j * 16, 16)], jnp.bfloat16)
                        b = plsc.bitcast(
                            rv[t, 1, c, pl.ds(j * 16, 16)], jnp.bfloat16)
                        v = a * wA[t] + b * wB[t]
                        x, y = plsc.unpack(v, format=FMT)
                        o[p] = x + y
                    ov[u * CT + c, pl.ds(j * 16, 16)] = plsc.bitcast(
                        plsc.pack(o[0], o[1], format=FMT), jnp.int32)

    load(0, 0)

    @pl.loop(0, NCHUNK, step=2)
    def _(k):
        for b in range(2):
            kk = k + b

            @pl.when(kk + 1 < NCHUNK)
            def _():
                load(kk + 1, 1 - b)

            wait_load(b)

            @pl.when(kk >= 2)
            def _():
                wait_store(b)

            compute(kk, b)
            store(kk, b)

    for b in range(2):
        wait_store(b)


def kernel(res, h_pre):
    xv = lax.bitcast_convert_type(res, jnp.uint16)
    lo = xv[:, 0:2, :].astype(jnp.uint32)
    hi = xv[:, 2:4, :].astype(jnp.uint32)
    w = lo | (hi << 16)
    res_i = lax.bitcast_convert_type(w, jnp.int32).reshape(M, 2, CT, 128)

    hb = h_pre.astype(jnp.bfloat16)
    hq = jnp.stack([hb[:, 0:2], hb[:, 2:4]], axis=2)
    wq = lax.bitcast_convert_type(hq, jnp.int32)
    wqr = wq.reshape(M // 64, 128)

    mesh = plsc.VectorSubcoreMesh(core_axis_name="c", subcore_axis_name="s")
    f = pl.kernel(
        _sc_body,
        out_type=jax.ShapeDtypeStruct((M * 8, 128), jnp.int32),
        mesh=mesh,
        compiler_params=pltpu.CompilerParams(
            needs_layout_passes=False, use_tc_tiling_on_sc=False),
        scratch_types=[
            pltpu.VMEM((RW // 64, 128), jnp.int32),
            pltpu.VMEM((T, 2, CT, 128), jnp.int32),
            pltpu.VMEM((T, 2, CT, 128), jnp.int32),
            pltpu.VMEM((OPR, 128), jnp.int32),
            pltpu.VMEM((OPR, 128), jnp.int32),
            pltpu.SemaphoreType.DMA,
            pltpu.SemaphoreType.DMA,
            pltpu.SemaphoreType.DMA,
            pltpu.SemaphoreType.DMA,
            pltpu.SemaphoreType.DMA,
        ],
    )
    oi = f(res_i, wqr)
    ou = lax.bitcast_convert_type(oi, jnp.uint32)
    olo = lax.bitcast_convert_type((ou & 0xFFFF).astype(jnp.uint16),
                                   jnp.bfloat16).reshape(M // 2, 1, D)
    ohi = lax.bitcast_convert_type((ou >> 16).astype(jnp.uint16),
                                   jnp.bfloat16).reshape(M // 2, 1, D)
    return jnp.concatenate([olo, ohi], axis=1).reshape(M, D)
